# Initial kernel scaffold; baseline (speedup 1.0000x reference)
import jax
import jax.numpy as jnp
from jax.experimental import pallas as pl


def kernel(node_features, edge_features, edge_index, W_node, b_node, W_edge, b_edge, W_mp, b_mp, W_r1, b_r1, W_r2, b_r2, W_r3, b_r3, W_ap, b_ap):
    raise NotImplementedError("write your pallas kernel here")


# speedup vs baseline: 1.8408x; 1.8408x over previous
import functools

import jax
import jax.numpy as jnp
from jax import lax
from jax.experimental import pallas as pl
from jax.experimental.pallas import tpu as pltpu
from jax.experimental.pallas import tpu_sc as plsc

N = 100000
H = 128
DE = 16
B = 2000
GRID = N // B
CH = 128
NFULL = N // CH
TAIL = N - NFULL * CH
NW = 32


def _tc_encode(nf, ef, W_node, bn, Wmpa0, Wmpc0, Wmpb, W_edge, be, bmp):
    def body(nf_ref, ef_ref, wn_ref, bn_ref, wa_ref, wc_ref, wb_ref, we_ref,
             be_ref, bmp_ref, n_ref, a_ref, c_ref, e0_ref, e1_ref, e2_ref):
        n0 = jnp.dot(nf_ref[...], wn_ref[...],
                     preferred_element_type=jnp.float32) + bn_ref[...]
        n_ref[...] = n0
        a_ref[...] = jnp.dot(n0, wa_ref[...], preferred_element_type=jnp.float32)
        c_ref[...] = jnp.dot(n0, wc_ref[...], preferred_element_type=jnp.float32)
        ef = ef_ref[...]
        for l, er in enumerate((e0_ref, e1_ref, e2_ref)):
            wb = wb_ref[l]
            wfold = jnp.dot(we_ref[...], wb, preferred_element_type=jnp.float32)
            bfold = jnp.dot(be_ref[...], wb,
                            preferred_element_type=jnp.float32) + bmp_ref[l]
            er[...] = jnp.dot(ef, wfold,
                              preferred_element_type=jnp.float32) + bfold

    row = lambda i: (i, 0)
    fixed = lambda i: (0, 0)
    fixed3 = lambda i: (0, 0, 0)
    out = jax.ShapeDtypeStruct((N, H), jnp.float32)
    return pl.pallas_call(
        body,
        grid=(GRID,),
        in_specs=[
            pl.BlockSpec((B, H), row),
            pl.BlockSpec((B, DE), row),
            pl.BlockSpec((H, H), fixed),
            pl.BlockSpec((1, H), fixed),
            pl.BlockSpec((H, H), fixed),
            pl.BlockSpec((H, H), fixed),
            pl.BlockSpec((3, H, H), fixed3),
            pl.BlockSpec((DE, H), fixed),
            pl.BlockSpec((1, H), fixed),
            pl.BlockSpec((3, 1, H), fixed3),
        ],
        out_specs=[pl.BlockSpec((B, H), row)] * 6,
        out_shape=[out] * 6,
        compiler_params=pltpu.CompilerParams(
            dimension_semantics=("parallel",)),
    )(nf, ef, W_node, bn, Wmpa0, Wmpc0, Wmpb, W_edge, be, bmp)


def _tc_proj(nodes, Wa, Wc):
    def body(n_ref, wa_ref, wc_ref, a_ref, c_ref):
        n = n_ref[...]
        a_ref[...] = jnp.dot(n, wa_ref[...], preferred_element_type=jnp.float32)
        c_ref[...] = jnp.dot(n, wc_ref[...], preferred_element_type=jnp.float32)

    row = lambda i: (i, 0)
    fixed = lambda i: (0, 0)
    out = jax.ShapeDtypeStruct((N, H), jnp.float32)
    return pl.pallas_call(
        body,
        grid=(GRID,),
        in_specs=[
            pl.BlockSpec((B, H), row),
            pl.BlockSpec((H, H), fixed),
            pl.BlockSpec((H, H), fixed),
        ],
        out_specs=[pl.BlockSpec((B, H), row)] * 2,
        out_shape=[out] * 2,
        compiler_params=pltpu.CompilerParams(
            dimension_semantics=("parallel",)),
    )(nodes, Wa, Wc)


def _sc_layer(A, C, E, nodes, src, dst):
    mesh = plsc.VectorSubcoreMesh(core_axis_name="c", subcore_axis_name="s")

    @functools.partial(
        pl.kernel,
        out_type=jax.ShapeDtypeStruct((N, H), jnp.float32),
        mesh=mesh,
        scratch_types=[
            pltpu.VMEM((CH,), jnp.int32),
            pltpu.VMEM((CH,), jnp.int32),
            pltpu.VMEM((CH, H), jnp.float32),
            pltpu.VMEM((CH, H), jnp.float32),
            pltpu.VMEM((CH, H), jnp.float32),
            pltpu.VMEM((CH, H), jnp.float32),
            pltpu.SemaphoreType.DMA,
            pltpu.SemaphoreType.DMA,
            pltpu.SemaphoreType.DMA,
        ],
    )
    def k(a_hbm, c_hbm, e_hbm, n_hbm, src_hbm, dst_hbm, out_hbm,
          si, di, ga, gc, ev, nv, sem_i, sem_a, sem_c):
        wid = lax.axis_index("s") * 2 + lax.axis_index("c")

        def do_chunk(base, ch):
            rows = pl.ds(0, ch)
            cp_s = pltpu.async_copy(src_hbm.at[pl.ds(base, ch)],
                                    si.at[rows], sem_i)
            cp_d = pltpu.async_copy(dst_hbm.at[pl.ds(base, ch)],
                                    di.at[rows], sem_i)
            cp_s.wait()
            cp_d.wait()
            cp_a = pltpu.async_copy(a_hbm.at[si.at[rows]], ga.at[rows], sem_a)
            cp_c = pltpu.async_copy(c_hbm.at[di.at[rows]], gc.at[rows], sem_c)
            cp_e = pltpu.async_copy(e_hbm.at[pl.ds(base, ch)],
                                    ev.at[rows], sem_i)
            cp_n = pltpu.async_copy(n_hbm.at[pl.ds(base, ch)],
                                    nv.at[rows], sem_i)
            cp_a.wait()
            cp_c.wait()
            cp_e.wait()
            cp_n.wait()

            @pl.loop(0, ch)
            def _(r):
                for j in range(H // 16):
                    sl = pl.ds(j * 16, 16)
                    m = ga[r, sl] + gc[r, sl] + ev[r, sl]
                    ev[r, sl] = nv[r, sl] + jnp.maximum(m, 0.0)

            pltpu.sync_copy(ev.at[rows], out_hbm.at[pl.ds(base, ch)])

        @pl.loop(wid, NFULL, step=NW)
        def _(ci):
            do_chunk(ci * CH, CH)

        @pl.when(wid == NW - 1)
        def _():
            do_chunk(NFULL * CH, TAIL)

    return k(A, C, E, nodes, src, dst)


def _tc_readout(nodes, W_r1, b1, W_r2, b2, w3row, b3, W_ap, bap):
    def body(n_ref, w1_ref, b1_ref, w2_ref, b2_ref, w3_ref, b3_ref,
             wap_ref, bap_ref, risk_ref, ap_ref):
        n = n_ref[...]
        h = jnp.maximum(jnp.dot(n, w1_ref[...],
                                preferred_element_type=jnp.float32)
                        + b1_ref[...], 0.0)
        h = jnp.maximum(jnp.dot(h, w2_ref[...],
                                preferred_element_type=jnp.float32)
                        + b2_ref[...], 0.0)
        r = jnp.sum(h * w3_ref[...], axis=1, keepdims=True) + b3_ref[...]
        risk_ref[...] = jax.nn.sigmoid(r)
        ap_ref[...] = jnp.dot(n, wap_ref[...],
                              preferred_element_type=jnp.float32) + bap_ref[...]

    row = lambda i: (i, 0)
    fixed = lambda i: (0, 0)
    return pl.pallas_call(
        body,
        grid=(GRID,),
        in_specs=[
            pl.BlockSpec((B, H), row),
            pl.BlockSpec((H, H), fixed),
            pl.BlockSpec((1, H), fixed),
            pl.BlockSpec((H, 64), fixed),
            pl.BlockSpec((1, 64), fixed),
            pl.BlockSpec((1, 64), fixed),
            pl.BlockSpec((1, 1), fixed),
            pl.BlockSpec((H, H), fixed),
            pl.BlockSpec((1, H), fixed),
        ],
        out_specs=[pl.BlockSpec((B, 1), row), pl.BlockSpec((B, H), row)],
        out_shape=[jax.ShapeDtypeStruct((N, 1), jnp.float32),
                   jax.ShapeDtypeStruct((N, H), jnp.float32)],
        compiler_params=pltpu.CompilerParams(
            dimension_semantics=("parallel",)),
    )(nodes, W_r1, b1, W_r2, b2, w3row, b3, W_ap, bap)


def kernel(node_features, edge_features, edge_index, W_node, b_node, W_edge,
           b_edge, W_mp, b_mp, W_r1, b_r1, W_r2, b_r2, W_r3, b_r3, W_ap, b_ap):
    src = edge_index[0]
    dst = edge_index[1]
    Wmpa = W_mp[:, 0:H, :]
    Wmpb = W_mp[:, H:2 * H, :]
    Wmpc = W_mp[:, 2 * H:3 * H, :]

    nodes, A, C, E0, E1, E2 = _tc_encode(
        node_features, edge_features, W_node, b_node.reshape(1, H),
        Wmpa[0], Wmpc[0], Wmpb, W_edge, b_edge.reshape(1, H),
        b_mp.reshape(3, 1, H))

    for l, E in enumerate((E0, E1, E2)):
        nodes = _sc_layer(A, C, E, nodes, src, dst)
        if l < 2:
            A, C = _tc_proj(nodes, Wmpa[l + 1], Wmpc[l + 1])

    risk, ap = _tc_readout(
        nodes, W_r1, b_r1.reshape(1, H), W_r2, b_r2.reshape(1, 64),
        W_r3.reshape(1, 64), b_r3.reshape(1, 1), W_ap, b_ap.reshape(1, H))
    return (risk, ap)
